# baseline (speedup 1.0000x reference)
import functools

import jax
import jax.numpy as jnp
from jax import lax
from jax.experimental import pallas as pl
from jax.experimental.pallas import tpu as pltpu
from jax.experimental.pallas import tpu_sc as plsc

_NC = 2
_NS = 16
_NW = _NC * _NS
_CHUNK = 128


def _xw_body(x_ref, w_ref, b_ref, o_ref):
    o_ref[...] = (
        jnp.dot(x_ref[...], w_ref[...], preferred_element_type=jnp.float32)
        + b_ref[...]
    )


def _compute_xw(x, W, b2d, rows_blk):
    n, e = x.shape
    g5h = W.shape[1]
    return pl.pallas_call(
        _xw_body,
        grid=(n // rows_blk,),
        in_specs=[
            pl.BlockSpec((rows_blk, e), lambda i: (i, 0)),
            pl.BlockSpec((e, g5h), lambda i: (0, 0)),
            pl.BlockSpec((1, g5h), lambda i: (0, 0)),
        ],
        out_specs=pl.BlockSpec((rows_blk, g5h), lambda i: (i, 0)),
        out_shape=jax.ShapeDtypeStruct((n, g5h), jnp.float32),
    )(x, W, b2d)


def _step_body(L, slotL, rows_blk, H,
               slot_ref, xw_ref, hcl_ref, hcr_ref,
               ul_ref, ur_ref, hc_in, hc_out,
               hcbuf, sem):
    del hc_in
    hl = hcl_ref[:, :H]
    cl = hcl_ref[:, H:]
    hr = hcr_ref[:, :H]
    cr = hcr_ref[:, H:]
    gates = (
        xw_ref[...]
        + jnp.dot(hl, ul_ref[...], preferred_element_type=jnp.float32)
        + jnp.dot(hr, ur_ref[...], preferred_element_type=jnp.float32)
    )
    i_g = gates[:, 0 * H:1 * H]
    fl_g = gates[:, 1 * H:2 * H]
    fr_g = gates[:, 2 * H:3 * H]
    o_g = gates[:, 3 * H:4 * H]
    u_g = gates[:, 4 * H:5 * H]
    c_new = (
        jax.nn.sigmoid(i_g) * jnp.tanh(u_g)
        + jax.nn.sigmoid(fl_g) * cl
        + jax.nn.sigmoid(fr_g) * cr
    )
    h_new = jax.nn.sigmoid(o_g) * jnp.tanh(c_new)
    hcbuf[...] = jnp.concatenate([h_new, c_new], axis=1)

    i = pl.program_id(0)
    per_b = L // rows_blk
    bb = i // per_b
    t = i % per_b
    dst = bb * slotL + slot_ref[0] * L + t * rows_blk
    cp = pltpu.make_async_copy(hcbuf, hc_out.at[pl.ds(dst, rows_blk), :], sem)
    cp.start()
    cp.wait()


def _tc_step(half, slot, xw, hcl, hcr, U_l, U_r, HC, L, slotL, rows_blk, H):
    n = hcl.shape[0]
    g5h = 5 * H
    nblk = n // rows_blk
    body = functools.partial(_step_body, L, slotL, rows_blk, H)
    hbm = pl.BlockSpec(memory_space=pltpu.MemorySpace.HBM)
    return pl.pallas_call(
        body,
        grid=(nblk,),
        in_specs=[
            pl.BlockSpec(memory_space=pltpu.MemorySpace.SMEM),
            pl.BlockSpec((rows_blk, g5h), lambda i: (half * nblk + i, 0)),
            pl.BlockSpec((rows_blk, 2 * H), lambda i: (i, 0)),
            pl.BlockSpec((rows_blk, 2 * H), lambda i: (i, 0)),
            pl.BlockSpec((H, g5h), lambda i: (0, 0)),
            pl.BlockSpec((H, g5h), lambda i: (0, 0)),
            hbm,
        ],
        out_specs=hbm,
        out_shape=jax.ShapeDtypeStruct(HC.shape, jnp.float32),
        scratch_shapes=[
            pltpu.VMEM((rows_blk, 2 * H), jnp.float32),
            pltpu.SemaphoreType.DMA,
        ],
        input_output_aliases={6: 0},
    )(slot, xw, hcl, hcr, U_l, U_r, HC)


def _sc_mesh():
    return plsc.VectorSubcoreMesh(core_axis_name="c", subcore_axis_name="s")


def _sc_gather_step_body(nchunk, rpw,
                         hc, gl, gr, hcl_out, hcr_out,
                         idx_v, buf, semo, *semg):
    wid = lax.axis_index("s") * _NC + lax.axis_index("c")
    base = wid * rpw
    nu = 2 * nchunk
    nb = nu - 1
    pltpu.sync_copy(gl.at[wid], idx_v.at[pl.ds(0, nchunk)])
    pltpu.sync_copy(gr.at[wid], idx_v.at[pl.ds(nchunk, nchunk)])
    outs = (hcl_out, hcr_out)
    gcp = [None] * nu
    ocp = [None] * nu

    def fire(u):
        gcp[u] = pltpu.async_copy(hc.at[idx_v.at[u]], buf.at[u % nb],
                                  semg[u % nb])

    for u in range(nb):
        fire(u)
    for u in range(nu):
        gcp[u].wait()
        side, jj = divmod(u, nchunk)
        dst = outs[side].at[pl.ds(base + jj * _CHUNK, _CHUNK), :]
        ocp[u] = pltpu.async_copy(buf.at[u % nb], dst, semo)
        if u + nb < nu:
            ocp[u].wait()
            ocp[u] = None
            fire(u + nb)
    for u in range(nu):
        if ocp[u] is not None:
            ocp[u].wait()


def _sc_gather_step(HC, gl, gr, rows, H):
    rpw = rows // _NW
    nchunk = rpw // _CHUNK
    nb = 2 * nchunk - 1
    body = functools.partial(_sc_gather_step_body, nchunk, rpw)
    out = jax.ShapeDtypeStruct((rows, 2 * H), jnp.float32)
    fn = pl.kernel(
        body,
        out_type=[out, out],
        mesh=_sc_mesh(),
        scratch_types=[
            pltpu.VMEM((2 * nchunk, _CHUNK), jnp.int32),
            pltpu.VMEM((nb, _CHUNK, 2 * H), jnp.float32),
            pltpu.SemaphoreType.DMA,
        ] + [pltpu.SemaphoreType.DMA] * nb,
    )
    return fn(HC, gl, gr)


def _final_half(hc, gfin_r, w_local, nchunk, buf, idx_v, sem, hc_out, base):
    pltpu.sync_copy(gfin_r.at[w_local], idx_v)
    copies = [
        pltpu.async_copy(
            hc.at[idx_v.at[j]],
            buf.at[pl.ds(j * _CHUNK, _CHUNK), :],
            sem,
        )
        for j in range(nchunk)
    ]
    for cp in copies:
        cp.wait()
    pltpu.sync_copy(buf, hc_out.at[base, :])


def _root_half(hc, gfl_src, tp_v, gfl_v, rbuf, sem, root_out, bb, L):
    pltpu.sync_copy(gfl_src, gfl_v)

    acc = jnp.full((16,), 0, jnp.int32)

    @pl.loop(0, L // 16, init_carry=acc)
    def mx(v, acc):
        return jnp.maximum(acc, tp_v[pl.ds(v * 16, 16)])

    root = lax.reduce_max(mx, (0,))
    ridx = jnp.broadcast_to(root, (16,))
    rowv = plsc.load_gather(gfl_v, [ridx])
    pltpu.async_copy(hc.at[rowv], rbuf, sem).wait()
    pltpu.sync_copy(rbuf.at[pl.ds(0, 1)], root_out.at[pl.ds(bb, 1)])


def _sc_gather_final_body(nchunk, rpw, nroot, L, S, hw,
                          hca, hcb, gfa, gfb, gfa_flat, gfb_flat, tree_p,
                          hc_out, root_out,
                          idx_v, buf, gfl_v, tp_v, rbuf, sem):
    wid = lax.axis_index("s") * _NC + lax.axis_index("c")
    base = pl.ds(wid * rpw, rpw)
    wpb = _NW // nroot
    bb = wid // wpb

    @pl.when(wid < hw)
    def _():
        _final_half(hca, gfa, wid, nchunk, buf, idx_v, sem, hc_out, base)

    @pl.when(wid >= hw)
    def _():
        _final_half(hcb, gfb, wid - hw, nchunk, buf, idx_v, sem, hc_out, base)

    @pl.when((wid % wpb == 0) & (wid < hw))
    def _():
        pltpu.sync_copy(tree_p.at[bb, S - 1], tp_v)
        _root_half(hca, gfa_flat.at[bb], tp_v, gfl_v, rbuf, sem,
                   root_out, bb, L)

    @pl.when((wid % wpb == 0) & (wid >= hw))
    def _():
        pltpu.sync_copy(tree_p.at[bb, S - 1], tp_v)
        _root_half(hcb, gfb_flat.at[bb - nroot // 2], tp_v, gfl_v, rbuf, sem,
                   root_out, bb, L)


def _sc_gather_final(HCA, HCB, gfa, gfb, gfa_flat, gfb_flat, tree_ids,
                     rows, H, L, S, nroot):
    rpw = rows // _NW
    nchunk = rpw // _CHUNK
    hw = _NW // 2
    body = functools.partial(_sc_gather_final_body, nchunk, rpw, nroot, L, S,
                             hw)
    fn = pl.kernel(
        body,
        compiler_params=pltpu.CompilerParams(needs_layout_passes=False),
        out_type=[
            jax.ShapeDtypeStruct((rows, 2 * H), jnp.float32),
            jax.ShapeDtypeStruct((nroot, 2 * H), jnp.float32),
        ],
        mesh=_sc_mesh(),
        scratch_types=[
            pltpu.VMEM((nchunk, _CHUNK), jnp.int32),
            pltpu.VMEM((rpw, 2 * H), jnp.float32),
            pltpu.VMEM((L,), jnp.int32),
            pltpu.VMEM((L,), jnp.int32),
            pltpu.VMEM((16, 2 * H), jnp.float32),
            pltpu.SemaphoreType.DMA,
        ],
    )
    return fn(HCA, HCB, gfa, gfb, gfa_flat, gfb_flat, tree_ids)


def _sc_index_body(B, Bh, L, S, slotL,
                   tree_p, tree_r, tree_l,
                   gla_out, gra_out, glb_out, grb_out, gfa_out, gfb_out,
                   ptr_v, il_v, ir_v, ip_v, gl_v, gr_v):
    wid = lax.axis_index("s") * _NC + lax.axis_index("c")
    lane = lax.broadcasted_iota(jnp.int32, (16,), 0)

    @pl.when(wid < B)
    def _():
        base = (wid % Bh) * slotL

        @pl.loop(0, L // 16)
        def init(v):
            ptr_v[pl.ds(v * 16, 16)] = base + v * 16 + lane

        for s in range(S):
            pltpu.sync_copy(tree_l.at[wid, s], il_v)
            pltpu.sync_copy(tree_r.at[wid, s], ir_v)
            pltpu.sync_copy(tree_p.at[wid, s], ip_v)

            @pl.loop(0, L // 16)
            def gat(v):
                sl = pl.ds(v * 16, 16)
                gl_v[sl] = plsc.load_gather(ptr_v, [il_v[sl]])
                gr_v[sl] = plsc.load_gather(ptr_v, [ir_v[sl]])

            @pl.when(wid < Bh)
            def _():
                pltpu.sync_copy(gl_v, gla_out.at[s, wid])
                pltpu.sync_copy(gr_v, gra_out.at[s, wid])

            @pl.when(wid >= Bh)
            def _():
                pltpu.sync_copy(gl_v, glb_out.at[s, wid - Bh])
                pltpu.sync_copy(gr_v, grb_out.at[s, wid - Bh])

            voff = base + (s + 1) * L

            @pl.loop(0, L // 16)
            def sca(v):
                ip = ip_v[pl.ds(v * 16, 16)]
                vals = voff + v * 16 + lane
                for k in range(16):
                    plsc.store_scatter(ptr_v, [ip], vals,
                                       mask=lane == k)

        @pl.when(wid < Bh)
        def _():
            pltpu.sync_copy(ptr_v, gfa_out.at[wid])

        @pl.when(wid >= Bh)
        def _():
            pltpu.sync_copy(ptr_v, gfb_out.at[wid - Bh])


def _sc_index(tree_ids, tree_ids_r, tree_ids_l, B, Bh, L, S, slotL):
    body = functools.partial(_sc_index_body, B, Bh, L, S, slotL)
    half = jax.ShapeDtypeStruct((S, Bh, L), jnp.int32)
    fin = jax.ShapeDtypeStruct((Bh, L), jnp.int32)
    fn = pl.kernel(
        body,
        compiler_params=pltpu.CompilerParams(needs_layout_passes=False),
        out_type=[half, half, half, half, fin, fin],
        mesh=_sc_mesh(),
        scratch_types=[pltpu.VMEM((L,), jnp.int32)] * 6,
    )
    return fn(tree_ids, tree_ids_r, tree_ids_l)


def kernel(input_ids, tree_ids, tree_ids_r, tree_ids_l, W, U_l, U_r, b):
    B, L, E = input_ids.shape
    H = U_l.shape[0]
    S = tree_ids.shape[1]
    slotL = (S + 1) * L
    rows = B * L
    Bh = B // 2
    rows_h = Bh * L
    nchunk_h = rows_h // _NW // _CHUNK

    GlA, GrA, GlB, GrB, gfa_flat, gfb_flat = _sc_index(
        tree_ids, tree_ids_r, tree_ids_l, B, Bh, L, S, slotL)
    glA = [GlA[s].reshape(_NW, nchunk_h, _CHUNK) for s in range(S)]
    grA = [GrA[s].reshape(_NW, nchunk_h, _CHUNK) for s in range(S)]
    glB = [GlB[s].reshape(_NW, nchunk_h, _CHUNK) for s in range(S)]
    grB = [GrB[s].reshape(_NW, nchunk_h, _CHUNK) for s in range(S)]
    hw = _NW // 2
    nchunk_f = rows // _NW // _CHUNK
    gfa = gfa_flat.reshape(hw, nchunk_f, _CHUNK)
    gfb = gfb_flat.reshape(hw, nchunk_f, _CHUNK)

    rows_blk = 2048
    xw = _compute_xw(input_ids.reshape(rows, E), W, b.reshape(1, 5 * H),
                     rows_blk)
    HCA = jnp.zeros((Bh * slotL, 2 * H), jnp.float32)
    HCB = jnp.zeros((Bh * slotL, 2 * H), jnp.float32)
    for s in range(S):
        hclA, hcrA = _sc_gather_step(HCA, glA[s], grA[s], rows_h, H)
        hclB, hcrB = _sc_gather_step(HCB, glB[s], grB[s], rows_h, H)
        slot = jnp.full((1,), s + 1, jnp.int32)
        HCA = _tc_step(0, slot, xw, hclA, hcrA, U_l, U_r, HCA,
                       L, slotL, rows_blk, H)
        HCB = _tc_step(1, slot, xw, hclB, hcrB, U_l, U_r, HCB,
                       L, slotL, rows_blk, H)
    hc_fin, root_fin = _sc_gather_final(HCA, HCB, gfa, gfb,
                                        gfa_flat, gfb_flat, tree_ids,
                                        rows, H, L, S, B)
    h = hc_fin[:, :H].reshape(B, L, H)
    c = hc_fin[:, H:].reshape(B, L, H)
    h_root = root_fin[:, :H]
    return (h, c, h_root)


# speedup vs baseline: 1.0279x; 1.0279x over previous
import functools

import jax
import jax.numpy as jnp
from jax import lax
from jax.experimental import pallas as pl
from jax.experimental.pallas import tpu as pltpu
from jax.experimental.pallas import tpu_sc as plsc

_NC = 2
_NS = 16
_NW = _NC * _NS
_CHUNK = 128


def _xw_body(x_ref, w_ref, b_ref, o_ref):
    o_ref[...] = (
        jnp.dot(x_ref[...], w_ref[...], preferred_element_type=jnp.float32)
        + b_ref[...]
    ).astype(jnp.bfloat16)


def _compute_xw(x, W, b2d, rows_blk):
    n, e = x.shape
    g5h = W.shape[1]
    return pl.pallas_call(
        _xw_body,
        grid=(n // rows_blk,),
        in_specs=[
            pl.BlockSpec((rows_blk, e), lambda i: (i, 0)),
            pl.BlockSpec((e, g5h), lambda i: (0, 0)),
            pl.BlockSpec((1, g5h), lambda i: (0, 0)),
        ],
        out_specs=pl.BlockSpec((rows_blk, g5h), lambda i: (i, 0)),
        out_shape=jax.ShapeDtypeStruct((n, g5h), jnp.bfloat16),
    )(x, W, b2d)


def _step_body(L, slotL, rows_blk, H,
               slot_ref, xw_ref, hcl_ref, hcr_ref,
               ul_ref, ur_ref, hc_in, hc_out,
               hcbuf, sem):
    del hc_in
    hl = hcl_ref[:, :H]
    cl = hcl_ref[:, H:]
    hr = hcr_ref[:, :H]
    cr = hcr_ref[:, H:]
    gates = (
        xw_ref[...].astype(jnp.float32)
        + jnp.dot(hl, ul_ref[...], preferred_element_type=jnp.float32)
        + jnp.dot(hr, ur_ref[...], preferred_element_type=jnp.float32)
    )
    i_g = gates[:, 0 * H:1 * H]
    fl_g = gates[:, 1 * H:2 * H]
    fr_g = gates[:, 2 * H:3 * H]
    o_g = gates[:, 3 * H:4 * H]
    u_g = gates[:, 4 * H:5 * H]
    c_new = (
        jax.nn.sigmoid(i_g) * jnp.tanh(u_g)
        + jax.nn.sigmoid(fl_g) * cl
        + jax.nn.sigmoid(fr_g) * cr
    )
    h_new = jax.nn.sigmoid(o_g) * jnp.tanh(c_new)
    hcbuf[...] = jnp.concatenate([h_new, c_new], axis=1)

    i = pl.program_id(0)
    per_b = L // rows_blk
    bb = i // per_b
    t = i % per_b
    dst = bb * slotL + slot_ref[0] * L + t * rows_blk
    cp = pltpu.make_async_copy(hcbuf, hc_out.at[pl.ds(dst, rows_blk), :], sem)
    cp.start()
    cp.wait()


def _tc_step(slot, xw, hcl, hcr, U_l, U_r, HC, L, slotL, rows_blk, H):
    n = xw.shape[0]
    g5h = 5 * H
    body = functools.partial(_step_body, L, slotL, rows_blk, H)
    hbm = pl.BlockSpec(memory_space=pltpu.MemorySpace.HBM)
    return pl.pallas_call(
        body,
        grid=(n // rows_blk,),
        in_specs=[
            pl.BlockSpec(memory_space=pltpu.MemorySpace.SMEM),
            pl.BlockSpec((rows_blk, g5h), lambda i: (i, 0)),
            pl.BlockSpec((rows_blk, 2 * H), lambda i: (i, 0)),
            pl.BlockSpec((rows_blk, 2 * H), lambda i: (i, 0)),
            pl.BlockSpec((H, g5h), lambda i: (0, 0)),
            pl.BlockSpec((H, g5h), lambda i: (0, 0)),
            hbm,
        ],
        out_specs=hbm,
        out_shape=jax.ShapeDtypeStruct(HC.shape, jnp.float32),
        scratch_shapes=[
            pltpu.VMEM((rows_blk, 2 * H), jnp.float32),
            pltpu.SemaphoreType.DMA,
        ],
        input_output_aliases={6: 0},
    )(slot, xw, hcl, hcr, U_l, U_r, HC)


def _sc_mesh():
    return plsc.VectorSubcoreMesh(core_axis_name="c", subcore_axis_name="s")


def _gather_chunk(table_ref, idx_v, j, buf, sem):
    return pltpu.async_copy(
        table_ref.at[idx_v.at[j]],
        buf.at[pl.ds(j * _CHUNK, _CHUNK), :],
        sem,
    )


def _sc_gather_step_body(nchunk, rpw,
                         hc, gl, gr, hcl_out, hcr_out,
                         idx_v, buf, semo, *semg):
    wid = lax.axis_index("s") * _NC + lax.axis_index("c")
    base = wid * rpw
    nu = 2 * nchunk
    nb = nu - 1
    pltpu.sync_copy(gl.at[wid], idx_v.at[pl.ds(0, nchunk)])
    pltpu.sync_copy(gr.at[wid], idx_v.at[pl.ds(nchunk, nchunk)])
    outs = (hcl_out, hcr_out)
    gcp = [None] * nu
    ocp = [None] * nu

    def fire(u):
        gcp[u] = pltpu.async_copy(hc.at[idx_v.at[u]], buf.at[u % nb],
                                  semg[u % nb])

    for u in range(nb):
        fire(u)
    for u in range(nu):
        gcp[u].wait()
        side, jj = divmod(u, nchunk)
        dst = outs[side].at[pl.ds(base + jj * _CHUNK, _CHUNK), :]
        ocp[u] = pltpu.async_copy(buf.at[u % nb], dst, semo)
        if u + nb < nu:
            ocp[u].wait()
            ocp[u] = None
            fire(u + nb)
    for u in range(nu):
        if ocp[u] is not None:
            ocp[u].wait()


def _sc_gather_step(HC, gl, gr, rows, H):
    rpw = rows // _NW
    nchunk = rpw // _CHUNK
    nb = 2 * nchunk - 1
    body = functools.partial(_sc_gather_step_body, nchunk, rpw)
    out = jax.ShapeDtypeStruct((rows, 2 * H), jnp.float32)
    fn = pl.kernel(
        body,
        out_type=[out, out],
        mesh=_sc_mesh(),
        scratch_types=[
            pltpu.VMEM((2 * nchunk, _CHUNK), jnp.int32),
            pltpu.VMEM((nb, _CHUNK, 2 * H), jnp.float32),
            pltpu.SemaphoreType.DMA,
        ] + [pltpu.SemaphoreType.DMA] * nb,
    )
    return fn(HC, gl, gr)


def _sc_gather_final_body(nchunk, rpw, nroot, L, S,
                          hc, gfin, gfin_flat, tree_p, hc_out, root_out,
                          idx_v, buf, gfl_v, tp_v, rbuf, sem):
    wid = lax.axis_index("s") * _NC + lax.axis_index("c")
    base = wid * rpw
    pltpu.sync_copy(gfin.at[wid], idx_v)
    copies = [_gather_chunk(hc, idx_v, j, buf, sem) for j in range(nchunk)]
    for cp in copies:
        cp.wait()
    pltpu.sync_copy(buf, hc_out.at[pl.ds(base, rpw), :])

    wpb = _NW // nroot
    bb = wid // wpb

    @pl.when(wid % wpb == 0)
    def _():
        pltpu.sync_copy(gfin_flat.at[bb], gfl_v)
        pltpu.sync_copy(tree_p.at[bb, S - 1], tp_v)

        acc = jnp.full((16,), 0, jnp.int32)

        @pl.loop(0, L // 16, init_carry=acc)
        def mx(v, acc):
            return jnp.maximum(acc, tp_v[pl.ds(v * 16, 16)])

        acc = mx
        root = lax.reduce_max(acc, (0,))
        ridx = jnp.broadcast_to(root, (16,))
        rowv = plsc.load_gather(gfl_v, [ridx])
        pltpu.async_copy(hc.at[rowv], rbuf, sem).wait()
        pltpu.sync_copy(rbuf.at[pl.ds(0, 1)], root_out.at[pl.ds(bb, 1)])


def _sc_gather_final(HC, gfin, gfin_flat, tree_ids, rows, H, L, S, nroot):
    rpw = rows // _NW
    nchunk = rpw // _CHUNK
    body = functools.partial(_sc_gather_final_body, nchunk, rpw, nroot, L, S)
    fn = pl.kernel(
        body,
        compiler_params=pltpu.CompilerParams(needs_layout_passes=False),
        out_type=[
            jax.ShapeDtypeStruct((rows, 2 * H), jnp.float32),
            jax.ShapeDtypeStruct((nroot, 2 * H), jnp.float32),
        ],
        mesh=_sc_mesh(),
        scratch_types=[
            pltpu.VMEM((nchunk, _CHUNK), jnp.int32),
            pltpu.VMEM((rpw, 2 * H), jnp.float32),
            pltpu.VMEM((L,), jnp.int32),
            pltpu.VMEM((L,), jnp.int32),
            pltpu.VMEM((16, 2 * H), jnp.float32),
            pltpu.SemaphoreType.DMA,
        ],
    )
    return fn(HC, gfin, gfin_flat, tree_ids)


def _sc_index_body(B, L, S, slotL,
                   tree_p, tree_r, tree_l, gl_out, gr_out, gfin_out,
                   ptr_v, il_v, ir_v, ip_v, gl_v, gr_v):
    wid = lax.axis_index("s") * _NC + lax.axis_index("c")
    lane = lax.broadcasted_iota(jnp.int32, (16,), 0)

    @pl.when(wid < B)
    def _():
        base = wid * slotL

        @pl.loop(0, L // 16)
        def init(v):
            ptr_v[pl.ds(v * 16, 16)] = base + v * 16 + lane

        for s in range(S):
            pltpu.sync_copy(tree_l.at[wid, s], il_v)
            pltpu.sync_copy(tree_r.at[wid, s], ir_v)
            pltpu.sync_copy(tree_p.at[wid, s], ip_v)

            @pl.loop(0, L // 16)
            def gat(v):
                sl = pl.ds(v * 16, 16)
                gl_v[sl] = plsc.load_gather(ptr_v, [il_v[sl]])
                gr_v[sl] = plsc.load_gather(ptr_v, [ir_v[sl]])
            pltpu.sync_copy(gl_v, gl_out.at[s, wid])
            pltpu.sync_copy(gr_v, gr_out.at[s, wid])

            voff = base + (s + 1) * L

            @pl.loop(0, L // 16)
            def sca(v):
                ip = ip_v[pl.ds(v * 16, 16)]
                vals = voff + v * 16 + lane
                for k in range(16):
                    plsc.store_scatter(ptr_v, [ip], vals,
                                       mask=lane == k)

        pltpu.sync_copy(ptr_v, gfin_out.at[wid])


def _sc_index(tree_ids, tree_ids_r, tree_ids_l, B, L, S, slotL):
    body = functools.partial(_sc_index_body, B, L, S, slotL)
    fn = pl.kernel(
        body,
        compiler_params=pltpu.CompilerParams(needs_layout_passes=False),
        out_type=[
            jax.ShapeDtypeStruct((S, B, L), jnp.int32),
            jax.ShapeDtypeStruct((S, B, L), jnp.int32),
            jax.ShapeDtypeStruct((B, L), jnp.int32),
        ],
        mesh=_sc_mesh(),
        scratch_types=[
            pltpu.VMEM((L,), jnp.int32),
            pltpu.VMEM((L,), jnp.int32),
            pltpu.VMEM((L,), jnp.int32),
            pltpu.VMEM((L,), jnp.int32),
            pltpu.VMEM((L,), jnp.int32),
            pltpu.VMEM((L,), jnp.int32),
        ],
    )
    return fn(tree_ids, tree_ids_r, tree_ids_l)


def kernel(input_ids, tree_ids, tree_ids_r, tree_ids_l, W, U_l, U_r, b):
    B, L, E = input_ids.shape
    H = U_l.shape[0]
    S = tree_ids.shape[1]
    slotL = (S + 1) * L
    rows = B * L

    rpw = rows // _NW
    nchunk = rpw // _CHUNK

    gl_all, gr_all, gfin_flat = _sc_index(tree_ids, tree_ids_r, tree_ids_l,
                                          B, L, S, slotL)
    gl_steps = [gl_all[s].reshape(_NW, nchunk, _CHUNK) for s in range(S)]
    gr_steps = [gr_all[s].reshape(_NW, nchunk, _CHUNK) for s in range(S)]
    gfin = gfin_flat.reshape(_NW, nchunk, _CHUNK)

    rows_blk = 2048
    xw = _compute_xw(input_ids.reshape(rows, E), W, b.reshape(1, 5 * H),
                     rows_blk)
    HC = jnp.zeros((B * slotL, 2 * H), jnp.float32)
    for s in range(S):
        hcl, hcr = _sc_gather_step(HC, gl_steps[s], gr_steps[s], rows, H)
        slot = jnp.full((1,), s + 1, jnp.int32)
        HC = _tc_step(slot, xw, hcl, hcr, U_l, U_r, HC, L, slotL, rows_blk, H)
    hc_fin, root_fin = _sc_gather_final(HC, gfin, gfin_flat, tree_ids,
                                        rows, H, L, S, B)
    h = hc_fin[:, :H].reshape(B, L, H)
    c = hc_fin[:, H:].reshape(B, L, H)
    h_root = root_fin[:, :H]
    return (h, c, h_root)


# speedup vs baseline: 1.0612x; 1.0324x over previous
import functools

import jax
import jax.numpy as jnp
from jax import lax
from jax.experimental import pallas as pl
from jax.experimental.pallas import tpu as pltpu
from jax.experimental.pallas import tpu_sc as plsc

_NC = 2
_NS = 16
_NW = _NC * _NS
_CHUNK = 128


def _xw_body(x_ref, w_ref, b_ref, o_ref):
    o_ref[...] = (
        jnp.dot(x_ref[...], w_ref[...], preferred_element_type=jnp.float32)
        + b_ref[...]
    ).astype(jnp.bfloat16)


def _compute_xw(x, W, b2d, rows_blk):
    n, e = x.shape
    g5h = W.shape[1]
    return pl.pallas_call(
        _xw_body,
        grid=(n // rows_blk,),
        in_specs=[
            pl.BlockSpec((rows_blk, e), lambda i: (i, 0)),
            pl.BlockSpec((e, g5h), lambda i: (0, 0)),
            pl.BlockSpec((1, g5h), lambda i: (0, 0)),
        ],
        out_specs=pl.BlockSpec((rows_blk, g5h), lambda i: (i, 0)),
        out_shape=jax.ShapeDtypeStruct((n, g5h), jnp.bfloat16),
    )(x, W, b2d)


def _step_body(L, slotL, rows_blk, H,
               slot_ref, xw_ref, hcl_ref, hcr_ref,
               ul_ref, ur_ref, hc_in, hc_out,
               hcbuf, sem):
    del hc_in
    hl = hcl_ref[:, :H]
    cl = hcl_ref[:, H:]
    hr = hcr_ref[:, :H]
    cr = hcr_ref[:, H:]
    gates = (
        xw_ref[...].astype(jnp.float32)
        + jnp.dot(hl, ul_ref[...], preferred_element_type=jnp.float32)
        + jnp.dot(hr, ur_ref[...], preferred_element_type=jnp.float32)
    )
    i_g = gates[:, 0 * H:1 * H]
    fl_g = gates[:, 1 * H:2 * H]
    fr_g = gates[:, 2 * H:3 * H]
    o_g = gates[:, 3 * H:4 * H]
    u_g = gates[:, 4 * H:5 * H]
    c_new = (
        jax.nn.sigmoid(i_g) * jnp.tanh(u_g)
        + jax.nn.sigmoid(fl_g) * cl
        + jax.nn.sigmoid(fr_g) * cr
    )
    h_new = jax.nn.sigmoid(o_g) * jnp.tanh(c_new)
    hcbuf[...] = jnp.concatenate([h_new, c_new], axis=1)

    i = pl.program_id(0)
    per_b = L // rows_blk
    bb = i // per_b
    t = i % per_b
    dst = bb * slotL + slot_ref[0] * L + t * rows_blk
    cp = pltpu.make_async_copy(hcbuf, hc_out.at[pl.ds(dst, rows_blk), :], sem)
    cp.start()
    cp.wait()


def _tc_step(slot, xw, hcl, hcr, U_l, U_r, HC, L, slotL, rows_blk, H):
    n = xw.shape[0]
    g5h = 5 * H
    body = functools.partial(_step_body, L, slotL, rows_blk, H)
    hbm = pl.BlockSpec(memory_space=pltpu.MemorySpace.HBM)
    return pl.pallas_call(
        body,
        grid=(n // rows_blk,),
        in_specs=[
            pl.BlockSpec(memory_space=pltpu.MemorySpace.SMEM),
            pl.BlockSpec((rows_blk, g5h), lambda i: (i, 0)),
            pl.BlockSpec((rows_blk, 2 * H), lambda i: (i, 0)),
            pl.BlockSpec((rows_blk, 2 * H), lambda i: (i, 0)),
            pl.BlockSpec((H, g5h), lambda i: (0, 0)),
            pl.BlockSpec((H, g5h), lambda i: (0, 0)),
            hbm,
        ],
        out_specs=hbm,
        out_shape=jax.ShapeDtypeStruct(HC.shape, jnp.float32),
        scratch_shapes=[
            pltpu.VMEM((rows_blk, 2 * H), jnp.float32),
            pltpu.SemaphoreType.DMA,
        ],
        input_output_aliases={6: 0},
    )(slot, xw, hcl, hcr, U_l, U_r, HC)


def _sc_mesh():
    return plsc.VectorSubcoreMesh(core_axis_name="c", subcore_axis_name="s")


def _gather_chunk(table_ref, idx_v, j, buf, sem):
    return pltpu.async_copy(
        table_ref.at[idx_v.at[j]],
        buf.at[pl.ds(j * _CHUNK, _CHUNK), :],
        sem,
    )


def _sc_gather_step_body(nchunk, rpw,
                         hc, gl, gr, hcl_out, hcr_out,
                         idx_v, buf, semo, *semg):
    wid = lax.axis_index("s") * _NC + lax.axis_index("c")
    base = wid * rpw
    nu = 2 * nchunk
    nb = nu - 1
    pltpu.sync_copy(gl.at[wid], idx_v.at[pl.ds(0, nchunk)])
    pltpu.sync_copy(gr.at[wid], idx_v.at[pl.ds(nchunk, nchunk)])
    outs = (hcl_out, hcr_out)
    gcp = [None] * nu
    ocp = [None] * nu

    def fire(u):
        gcp[u] = pltpu.async_copy(hc.at[idx_v.at[u]], buf.at[u % nb],
                                  semg[u % nb])

    for u in range(nb):
        fire(u)
    for u in range(nu):
        gcp[u].wait()
        side, jj = divmod(u, nchunk)
        dst = outs[side].at[pl.ds(base + jj * _CHUNK, _CHUNK), :]
        ocp[u] = pltpu.async_copy(buf.at[u % nb], dst, semo)
        if u + nb < nu:
            ocp[u].wait()
            ocp[u] = None
            fire(u + nb)
    for u in range(nu):
        if ocp[u] is not None:
            ocp[u].wait()


def _sc_gather_step(HC, gl, gr, rows, H):
    rpw = rows // _NW
    nchunk = rpw // _CHUNK
    nb = 2 * nchunk - 1
    body = functools.partial(_sc_gather_step_body, nchunk, rpw)
    out = jax.ShapeDtypeStruct((rows, 2 * H), jnp.float32)
    fn = pl.kernel(
        body,
        out_type=[out, out],
        mesh=_sc_mesh(),
        scratch_types=[
            pltpu.VMEM((2 * nchunk, _CHUNK), jnp.int32),
            pltpu.VMEM((nb, _CHUNK, 2 * H), jnp.float32),
            pltpu.SemaphoreType.DMA,
        ] + [pltpu.SemaphoreType.DMA] * nb,
    )
    return fn(HC, gl, gr)


def _sc_gather_final_body(nchunk, rpw, nroot, L, S,
                          hc, gfin, gfin_flat, tree_p, hc_out, root_out,
                          idx_v, buf, gfl_v, tp_v, rbuf, sem):
    wid = lax.axis_index("s") * _NC + lax.axis_index("c")
    base = wid * rpw
    pltpu.sync_copy(gfin.at[wid], idx_v)
    copies = [_gather_chunk(hc, idx_v, j, buf, sem) for j in range(nchunk)]
    for cp in copies:
        cp.wait()
    pltpu.sync_copy(buf, hc_out.at[pl.ds(base, rpw), :])

    wpb = _NW // nroot
    bb = wid // wpb

    @pl.when(wid % wpb == 0)
    def _():
        pltpu.sync_copy(gfin_flat.at[bb], gfl_v)
        pltpu.sync_copy(tree_p.at[bb, S - 1], tp_v)

        acc = jnp.full((16,), 0, jnp.int32)

        @pl.loop(0, L // 16, init_carry=acc)
        def mx(v, acc):
            return jnp.maximum(acc, tp_v[pl.ds(v * 16, 16)])

        acc = mx
        root = lax.reduce_max(acc, (0,))
        ridx = jnp.broadcast_to(root, (16,))
        rowv = plsc.load_gather(gfl_v, [ridx])
        pltpu.async_copy(hc.at[rowv], rbuf, sem).wait()
        pltpu.sync_copy(rbuf.at[pl.ds(0, 1)], root_out.at[pl.ds(bb, 1)])


def _sc_gather_final(HC, gfin, gfin_flat, tree_ids, rows, H, L, S, nroot):
    rpw = rows // _NW
    nchunk = rpw // _CHUNK
    body = functools.partial(_sc_gather_final_body, nchunk, rpw, nroot, L, S)
    fn = pl.kernel(
        body,
        compiler_params=pltpu.CompilerParams(needs_layout_passes=False),
        out_type=[
            jax.ShapeDtypeStruct((rows, 2 * H), jnp.float32),
            jax.ShapeDtypeStruct((nroot, 2 * H), jnp.float32),
        ],
        mesh=_sc_mesh(),
        scratch_types=[
            pltpu.VMEM((nchunk, _CHUNK), jnp.int32),
            pltpu.VMEM((rpw, 2 * H), jnp.float32),
            pltpu.VMEM((L,), jnp.int32),
            pltpu.VMEM((L,), jnp.int32),
            pltpu.VMEM((16, 2 * H), jnp.float32),
            pltpu.SemaphoreType.DMA,
        ],
    )
    return fn(HC, gfin, gfin_flat, tree_ids)


def _sc_index_body(B, L, S, slotL,
                   tree_p, tree_r, tree_l, gl_out, gr_out, gfin_out,
                   ptr_v, il_v, ir_v, ip_v, gl_v, gr_v, sem_in, sem_out):
    wid = lax.axis_index("s") * _NC + lax.axis_index("c")
    lane = lax.broadcasted_iota(jnp.int32, (16,), 0)

    @pl.when(wid < B)
    def _():
        base = wid * slotL

        @pl.loop(0, L // 16)
        def init(v):
            ptr_v[pl.ds(v * 16, 16)] = base + v * 16 + lane

        def fire_in(s, p):
            return [
                pltpu.async_copy(tree_l.at[wid, s], il_v.at[p], sem_in),
                pltpu.async_copy(tree_r.at[wid, s], ir_v.at[p], sem_in),
                pltpu.async_copy(tree_p.at[wid, s], ip_v.at[p], sem_in),
            ]

        icp = fire_in(0, 0)
        ocp = [None, None]
        for s in range(S):
            p = s % 2
            for cp in icp:
                cp.wait()
            if s + 1 < S:
                icp = fire_in(s + 1, 1 - p)
            if ocp[p] is not None:
                for cp in ocp[p]:
                    cp.wait()
                ocp[p] = None

            @pl.loop(0, L // 16)
            def gat(v):
                sl = pl.ds(v * 16, 16)
                gl_v[p, sl] = plsc.load_gather(ptr_v, [il_v[p, sl]])
                gr_v[p, sl] = plsc.load_gather(ptr_v, [ir_v[p, sl]])

            ocp[p] = [
                pltpu.async_copy(gl_v.at[p], gl_out.at[s, wid], sem_out),
                pltpu.async_copy(gr_v.at[p], gr_out.at[s, wid], sem_out),
            ]
            voff = base + (s + 1) * L

            @pl.loop(0, L // 16)
            def sca(v):
                ip = ip_v[p, pl.ds(v * 16, 16)]
                vals = voff + v * 16 + lane
                for k in range(16):
                    plsc.store_scatter(ptr_v, [ip], vals,
                                       mask=lane == k)

        for o in ocp:
            if o is not None:
                for cp in o:
                    cp.wait()
        pltpu.sync_copy(ptr_v, gfin_out.at[wid])


def _sc_index(tree_ids, tree_ids_r, tree_ids_l, B, L, S, slotL):
    body = functools.partial(_sc_index_body, B, L, S, slotL)
    fn = pl.kernel(
        body,
        compiler_params=pltpu.CompilerParams(needs_layout_passes=False),
        out_type=[
            jax.ShapeDtypeStruct((S, B, L), jnp.int32),
            jax.ShapeDtypeStruct((S, B, L), jnp.int32),
            jax.ShapeDtypeStruct((B, L), jnp.int32),
        ],
        mesh=_sc_mesh(),
        scratch_types=[
            pltpu.VMEM((L,), jnp.int32),
            pltpu.VMEM((2, L), jnp.int32),
            pltpu.VMEM((2, L), jnp.int32),
            pltpu.VMEM((2, L), jnp.int32),
            pltpu.VMEM((2, L), jnp.int32),
            pltpu.VMEM((2, L), jnp.int32),
            pltpu.SemaphoreType.DMA,
            pltpu.SemaphoreType.DMA,
        ],
    )
    return fn(tree_ids, tree_ids_r, tree_ids_l)


def kernel(input_ids, tree_ids, tree_ids_r, tree_ids_l, W, U_l, U_r, b):
    B, L, E = input_ids.shape
    H = U_l.shape[0]
    S = tree_ids.shape[1]
    slotL = (S + 1) * L
    rows = B * L

    rpw = rows // _NW
    nchunk = rpw // _CHUNK

    gl_all, gr_all, gfin_flat = _sc_index(tree_ids, tree_ids_r, tree_ids_l,
                                          B, L, S, slotL)
    gl_steps = [gl_all[s].reshape(_NW, nchunk, _CHUNK) for s in range(S)]
    gr_steps = [gr_all[s].reshape(_NW, nchunk, _CHUNK) for s in range(S)]
    gfin = gfin_flat.reshape(_NW, nchunk, _CHUNK)

    rows_blk = 2048
    xw = _compute_xw(input_ids.reshape(rows, E), W, b.reshape(1, 5 * H),
                     rows_blk)
    HC = jnp.zeros((B * slotL, 2 * H), jnp.float32)
    for s in range(S):
        hcl, hcr = _sc_gather_step(HC, gl_steps[s], gr_steps[s], rows, H)
        slot = jnp.full((1,), s + 1, jnp.int32)
        HC = _tc_step(slot, xw, hcl, hcr, U_l, U_r, HC, L, slotL, rows_blk, H)
    hc_fin, root_fin = _sc_gather_final(HC, gfin, gfin_flat, tree_ids,
                                        rows, H, L, S, B)
    h = hc_fin[:, :H].reshape(B, L, H)
    c = hc_fin[:, H:].reshape(B, L, H)
    h_root = root_fin[:, :H]
    return (h, c, h_root)
